# baseline (speedup 1.0000x reference)
import dataclasses
import functools
import jax
import jax.numpy as jnp
from jax import lax
from jax.experimental import pallas as pl
from jax.experimental.pallas import tpu as pltpu
from jax.experimental.pallas import tpu_sc as plsc

N = 10000
E = 320000
R = 2
IN_DIM = 128
HID = 64
NEG = 0.2

NPAD = 10240
NSLICE = NPAD // 16
ROWS = 2560
EPAD = ROWS * 128
NTILES = 32
ROWS_PER_TILE = ROWS // NTILES
SLABS = ROWS_PER_TILE // 8

_mesh = plsc.VectorSubcoreMesh(core_axis_name="c", subcore_axis_name="s")

_sc_params = pltpu.CompilerParams()
if "needs_layout_passes" in pltpu.CompilerParams.__dataclass_fields__:
    _sc_params = dataclasses.replace(_sc_params, needs_layout_passes=False)
_sc_lin_params = _sc_params
if "use_tc_tiling_on_sc" in pltpu.CompilerParams.__dataclass_fields__:
    _sc_lin_params = dataclasses.replace(_sc_params, use_tc_tiling_on_sc=False)


def _lrelu(x):
    return jnp.where(x >= 0.0, x, NEG * x)


def _elu(x):
    return jnp.where(x > 0.0, x, jnp.exp(jnp.minimum(x, 0.0)) - 1.0)


_NB = 1000


def _head_block(x, w_ref, q_ref, k_ref, xw_ref, s_ref):
    cols = [None] * (2 * R)
    for r in range(R):
        xw = jnp.dot(x, w_ref[r], preferred_element_type=jnp.float32)
        xw_ref[r] = xw
        cols[r] = jnp.sum(xw * q_ref[...], axis=1, keepdims=True)
        cols[R + r] = jnp.sum(xw * k_ref[...], axis=1, keepdims=True)
    s_ref[...] = jnp.concatenate(cols, axis=1)


def _head1(x_ref, w_ref, q_ref, k_ref, xw_ref, s_ref):
    _head_block(x_ref[...], w_ref, q_ref, k_ref, xw_ref, s_ref)


def _finalize(p_ref, d_ref):
    d = d_ref[0] + d_ref[1] + 1e-16
    return _elu((p_ref[0] + p_ref[1]) / d)


def _head2(p_ref, d_ref, w_ref, q_ref, k_ref, xw_ref, s_ref):
    _head_block(_finalize(p_ref, d_ref), w_ref, q_ref, k_ref, xw_ref, s_ref)


def _ubound(s_ref, u_ref):
    u = None
    for r in range(R):
        kmax = jnp.max(s_ref[:, R + r:R + r + 1])
        b = _lrelu(s_ref[:, r:r + 1] + kmax)
        u = b if u is None else jnp.maximum(u, b)
    u_ref[...] = u


def _decoder(p_ref, d_ref, w_ref, b_ref, h2_ref, h3_ref):
    h2 = _finalize(p_ref, d_ref)
    h2_ref[...] = h2
    h3_ref[...] = (
        jnp.dot(h2, w_ref[...], preferred_element_type=jnp.float32)
        + b_ref[...]
    )


def _scale_body(g_ref, e_ref, o_ref):
    o_ref[...] = g_ref[...] * e_ref[...]


_head_out_shapes = (
    jax.ShapeDtypeStruct((R, N, HID), jnp.float32),
    jax.ShapeDtypeStruct((N, 2 * R), jnp.float32),
)
_head_out_specs = (
    pl.BlockSpec((R, _NB, HID), lambda i: (0, i, 0)),
    pl.BlockSpec((_NB, 2 * R), lambda i: (i, 0)),
)
_qk_spec = pl.BlockSpec((1, HID), lambda i: (0, 0))
_pd_specs = [
    pl.BlockSpec((R, _NB, HID), lambda i: (0, i, 0)),
    pl.BlockSpec((R, _NB, 1), lambda i: (0, i, 0)),
]


def _run_head1(x, w, qt, kt):
    return pl.pallas_call(
        _head1,
        grid=(N // _NB,),
        in_specs=[
            pl.BlockSpec((_NB, IN_DIM), lambda i: (i, 0)),
            pl.BlockSpec((R, IN_DIM, HID), lambda i: (0, 0, 0)),
            _qk_spec, _qk_spec,
        ],
        out_specs=_head_out_specs,
        out_shape=_head_out_shapes,
    )(x, w, qt, kt)


def _run_head2(parts, dens, w, qt, kt):
    return pl.pallas_call(
        _head2,
        grid=(N // _NB,),
        in_specs=_pd_specs + [
            pl.BlockSpec((R, HID, HID), lambda i: (0, 0, 0)),
            _qk_spec, _qk_spec,
        ],
        out_specs=_head_out_specs,
        out_shape=_head_out_shapes,
    )(parts, dens, w, qt, kt)


def _run_ubound(s):
    return pl.pallas_call(
        _ubound,
        out_shape=jax.ShapeDtypeStruct((N, 1), jnp.float32),
    )(s)


def _run_decoder(parts, dens, wt, b):
    return pl.pallas_call(
        _decoder,
        grid=(N // _NB,),
        in_specs=_pd_specs + [
            pl.BlockSpec((HID, IN_DIM), lambda i: (0, 0)),
            pl.BlockSpec((1, IN_DIM), lambda i: (0, 0)),
        ],
        out_specs=(
            pl.BlockSpec((_NB, HID), lambda i: (i, 0)),
            pl.BlockSpec((_NB, IN_DIM), lambda i: (i, 0)),
        ),
        out_shape=(
            jax.ShapeDtypeStruct((N, HID), jnp.float32),
            jax.ShapeDtypeStruct((N, IN_DIM), jnp.float32),
        ),
    )(parts, dens, wt, b)


_SCALE_BE = 8192


def _run_scale(gathered, ex_col):
    grid = EPAD // _SCALE_BE
    return pl.pallas_call(
        _scale_body,
        grid=(grid,),
        in_specs=[
            pl.BlockSpec((_SCALE_BE, HID), lambda i: (i, 0)),
            pl.BlockSpec((_SCALE_BE, 1), lambda i: (i, 0)),
        ],
        out_specs=pl.BlockSpec((_SCALE_BE, HID), lambda i: (i, 0)),
        out_shape=jax.ShapeDtypeStruct((EPAD, HID), jnp.float32),
    )(gathered, ex_col)


@functools.partial(
    pl.kernel,
    mesh=_mesh,
    compiler_params=_sc_params,
    out_type=(
        jax.ShapeDtypeStruct((ROWS, 128), jnp.float32),
        jax.ShapeDtypeStruct((2, NPAD), jnp.float32),
    ),
    scratch_types=[
        pltpu.VMEM((R * N,), jnp.float32),
        pltpu.VMEM((R * N,), jnp.float32),
        pltpu.VMEM((NPAD,), jnp.float32),
        pltpu.VMEM((ROWS_PER_TILE, 128), jnp.int32),
        pltpu.VMEM((ROWS_PER_TILE, 128), jnp.int32),
        pltpu.VMEM((ROWS_PER_TILE, 128), jnp.int32),
        pltpu.VMEM((ROWS_PER_TILE, 128), jnp.float32),
        pltpu.VMEM((NPAD,), jnp.float32),
        pltpu.VMEM((16, NSLICE), jnp.float32),
        pltpu.VMEM_SHARED((16, NPAD), jnp.float32),
        pltpu.SemaphoreType.DMA,
    ],
)
def _sc_scores(sq_h, sk_h, u_h, iq_h, ik_h, dd_h, z_h,
               ex_h, den_h,
               sq_v, sk_v, u_v, iq_v, ik_v, dd_v, ex_v,
               priv, red_v, stage, ld_sem):
    c = lax.axis_index("c")
    s = lax.axis_index("s")
    wid = s * 2 + c

    base = wid * ROWS_PER_TILE
    lds = [
        pltpu.async_copy(sq_h, sq_v, ld_sem),
        pltpu.async_copy(sk_h, sk_v, ld_sem),
        pltpu.async_copy(u_h, u_v, ld_sem),
        pltpu.async_copy(iq_h.at[pl.ds(base, ROWS_PER_TILE)], iq_v, ld_sem),
        pltpu.async_copy(ik_h.at[pl.ds(base, ROWS_PER_TILE)], ik_v, ld_sem),
        pltpu.async_copy(dd_h.at[pl.ds(base, ROWS_PER_TILE)], dd_v, ld_sem),
        pltpu.async_copy(z_h, priv, ld_sem),
    ]
    for h in lds:
        h.wait()

    @pl.loop(0, ROWS_PER_TILE)
    def _(j):
        @pl.loop(0, 8)
        def _(v):
            sl = pl.ds(v * 16, 16)
            dd = dd_v[j, sl]
            a = plsc.load_gather(sq_v, [iq_v[j, sl]])
            b = plsc.load_gather(sk_v, [ik_v[j, sl]])
            u = plsc.load_gather(u_v, [dd])
            e = jnp.exp(_lrelu(a + b) - u)
            ex_v[j, sl] = e
            plsc.addupdate_scatter(priv, [dd], e)

    ex_out = pltpu.async_copy(
        ex_v, ex_h.at[pl.ds(base, ROWS_PER_TILE)], ld_sem)
    pltpu.sync_copy(priv, stage.at[s])
    plsc.subcore_barrier()

    pltpu.sync_copy(stage.at[:, pl.ds(s * NSLICE, NSLICE)], red_v)

    @pl.loop(0, NSLICE // 16)
    def _(v):
        sl = pl.ds(v * 16, 16)
        acc = red_v[0, sl]
        for j in range(1, 16):
            acc = acc + red_v[j, sl]
        red_v[0, sl] = acc

    pltpu.sync_copy(red_v.at[0], den_h.at[c, pl.ds(s * NSLICE, NSLICE)])
    ex_out.wait()


@functools.partial(
    pl.kernel,
    mesh=_mesh,
    compiler_params=_sc_lin_params,
    out_type=jax.ShapeDtypeStruct((EPAD, HID), jnp.float32),
    scratch_types=[
        pltpu.VMEM((ROWS_PER_TILE, 128), jnp.int32),
        pltpu.VMEM((512, HID), jnp.float32),
        pltpu.VMEM((512, HID), jnp.float32),
        pltpu.SemaphoreType.DMA,
        pltpu.SemaphoreType.DMA,
        pltpu.SemaphoreType.DMA,
        pltpu.SemaphoreType.DMA,
    ],
)
def _sc_gather(xw_h, ik_h, g_h, ik_v, rows_a, rows_b,
               g_sem_a, g_sem_b, w_sem_a, w_sem_b):
    c = lax.axis_index("c")
    s = lax.axis_index("s")
    wid = s * 2 + c
    base = wid * ROWS_PER_TILE
    pltpu.sync_copy(ik_h.at[pl.ds(base, ROWS_PER_TILE)], ik_v)

    bufs = (rows_a, rows_b)
    g_sems = (g_sem_a, g_sem_b)
    w_sems = (w_sem_a, w_sem_b)
    n_chunks = ROWS_PER_TILE // 4
    w_hs = [None, None]
    for t in range(n_chunks):
        b = t % 2
        if w_hs[b] is not None:
            w_hs[b].wait()
        g_hs = [
            pltpu.async_copy(
                xw_h.at[ik_v.at[t * 4 + j]],
                bufs[b].at[pl.ds(j * 128, 128)],
                g_sems[b])
            for j in range(4)
        ]
        for h in g_hs:
            h.wait()
        w_hs[b] = pltpu.async_copy(
            bufs[b], g_h.at[pl.ds((base + t * 4) * 128, 512)], w_sems[b])
    for h in w_hs:
        if h is not None:
            h.wait()


@functools.partial(
    pl.kernel,
    mesh=_mesh,
    compiler_params=_sc_lin_params,
    out_type=jax.ShapeDtypeStruct((2, NPAD, HID), jnp.float32),
    scratch_types=[
        pltpu.VMEM((ROWS_PER_TILE, 128), jnp.int32),
        pltpu.VMEM((512, HID), jnp.float32),
        pltpu.VMEM((512, HID), jnp.float32),
        pltpu.VMEM_SHARED((NPAD, HID), jnp.float32),
        pltpu.SemaphoreType.DMA,
        pltpu.SemaphoreType.DMA,
        pltpu.SemaphoreType.DMA,
        pltpu.SemaphoreType.DMA,
    ],
)
def _sc_scatter(sc_h, dd_h, z_h, out_h, dd_v, rows_a, rows_b, racc,
                i_sem_a, i_sem_b, s_sem_a, s_sem_b):
    c = lax.axis_index("c")
    s = lax.axis_index("s")
    wid = s * 2 + c

    @pl.when(s == 0)
    def _():
        pltpu.sync_copy(z_h, racc)

    plsc.subcore_barrier()

    base = wid * ROWS_PER_TILE
    pltpu.sync_copy(dd_h.at[pl.ds(base, ROWS_PER_TILE)], dd_v)

    bufs = (rows_a, rows_b)
    i_sems = (i_sem_a, i_sem_b)
    s_sems = (s_sem_a, s_sem_b)
    n_chunks = ROWS_PER_TILE // 4

    def start_in(t):
        b = t % 2
        return pltpu.async_copy(
            sc_h.at[pl.ds((base + t * 4) * 128, 512)], bufs[b], i_sems[b])

    in_hs = [start_in(0), None]
    sc_hs = [[], []]
    for t in range(n_chunks):
        b = t % 2
        if t + 1 < n_chunks:
            nb = (t + 1) % 2
            for h in sc_hs[nb]:
                h.wait()
            sc_hs[nb] = []
            in_hs[nb] = start_in(t + 1)
        in_hs[b].wait()
        sc_hs[b] = [
            pltpu.async_copy(
                bufs[b].at[pl.ds(j * 128, 128)],
                racc.at[dd_v.at[t * 4 + j]],
                s_sems[b], add=True)
            for j in range(4)
        ]
    for hs in sc_hs:
        for h in hs:
            h.wait()

    plsc.subcore_barrier()

    @pl.when(s == 0)
    def _():
        pltpu.sync_copy(racc, out_h.at[c])


def _layer(xw, s, iqp, ikp, ddp, z1, z2):
    u = _run_ubound(s).reshape(N)
    u_pad = jnp.concatenate([u, jnp.full((NPAD - N,), 1000.0, jnp.float32)])
    sq_flat = s[:, :R].T.reshape(R * N)
    sk_flat = s[:, R:].T.reshape(R * N)
    ex, den = _sc_scores(sq_flat, sk_flat, u_pad, iqp, ikp, ddp, z1)
    gathered = _sc_gather(xw.reshape(R * N, HID), ikp)
    scaled = _run_scale(gathered, ex.reshape(EPAD, 1))
    parts = _sc_scatter(scaled, ddp, z2)
    return parts, den.reshape(R, NPAD, 1)


@jax.jit
def kernel(features, edge_index, edge_type, w1, q1, k1, w2, q2, k2,
           dec_w, dec_b):
    src = edge_index[0]
    dst = edge_index[1]
    iq = edge_type * N + dst
    ik = edge_type * N + src

    def pad_to_rows(a, v):
        return jnp.pad(a, (0, EPAD - E), constant_values=v).reshape(ROWS, 128)

    iqp = pad_to_rows(iq, 0)
    ikp = pad_to_rows(ik, 0)
    ddp = pad_to_rows(dst, N)
    z1 = jnp.zeros((NPAD,), jnp.float32)
    z2 = jnp.zeros((NPAD, HID), jnp.float32)

    xw1, s1 = _run_head1(features, w1, q1.T, k1.T)
    parts1, den1 = _layer(xw1, s1, iqp, ikp, ddp, z1, z2)

    xw2, s2 = _run_head2(parts1, den1, w2, q2.T, k2.T)
    parts2, den2 = _layer(xw2, s2, iqp, ikp, ddp, z1, z2)

    h2, h3 = _run_decoder(parts2, den2, dec_w.T, dec_b.reshape(1, IN_DIM))
    return (h2, h3)


# speedup vs baseline: 1.0123x; 1.0123x over previous
import dataclasses
import functools
import jax
import jax.numpy as jnp
from jax import lax
from jax.experimental import pallas as pl
from jax.experimental.pallas import tpu as pltpu
from jax.experimental.pallas import tpu_sc as plsc

N = 10000
E = 320000
R = 2
IN_DIM = 128
HID = 64
NEG = 0.2

NPAD = 10240
NSLICE = NPAD // 16
SROWS = 157
UROWS = NPAD // 128
ROWS = 2560
EPAD = ROWS * 128
NTILES = 32
ROWS_PER_TILE = ROWS // NTILES
SLABS = ROWS_PER_TILE // 8

_mesh = plsc.VectorSubcoreMesh(core_axis_name="c", subcore_axis_name="s")

_sc_params = pltpu.CompilerParams()
if "needs_layout_passes" in pltpu.CompilerParams.__dataclass_fields__:
    _sc_params = dataclasses.replace(_sc_params, needs_layout_passes=False)
_sc_lin_params = _sc_params
if "use_tc_tiling_on_sc" in pltpu.CompilerParams.__dataclass_fields__:
    _sc_lin_params = dataclasses.replace(_sc_params, use_tc_tiling_on_sc=False)


def _lrelu(x):
    return jnp.where(x >= 0.0, x, NEG * x)


def _elu(x):
    return jnp.where(x > 0.0, x, jnp.exp(jnp.minimum(x, 0.0)) - 1.0)


_NB = 1000


def _head_block(x, w_ref, q_ref, k_ref, xw_ref, s_ref):
    cols = [None] * (2 * R)
    for r in range(R):
        xw = jnp.dot(x, w_ref[r], preferred_element_type=jnp.float32)
        xw_ref[r] = xw
        cols[r] = jnp.sum(xw * q_ref[...], axis=1, keepdims=True)
        cols[R + r] = jnp.sum(xw * k_ref[...], axis=1, keepdims=True)
    s_ref[...] = jnp.concatenate(cols, axis=1)


def _head1(x_ref, w_ref, q_ref, k_ref, xw_ref, s_ref):
    _head_block(x_ref[...], w_ref, q_ref, k_ref, xw_ref, s_ref)


def _finalize(p_ref, d_ref):
    d = d_ref[0] + d_ref[1] + 1e-16
    return _elu((p_ref[0] + p_ref[1]) / d)


def _head2(p_ref, d_ref, w_ref, q_ref, k_ref, xw_ref, s_ref):
    _head_block(_finalize(p_ref, d_ref), w_ref, q_ref, k_ref, xw_ref, s_ref)


def _ubound(s_ref, u_ref):
    u = None
    for r in range(R):
        kmax = jnp.max(s_ref[:, R + r:R + r + 1])
        b = _lrelu(s_ref[:, r:r + 1] + kmax)
        u = b if u is None else jnp.maximum(u, b)
    u_ref[...] = u


def _decoder(p_ref, d_ref, w_ref, b_ref, h2_ref, h3_ref):
    h2 = _finalize(p_ref, d_ref)
    h2_ref[...] = h2
    h3_ref[...] = (
        jnp.dot(h2, w_ref[...], preferred_element_type=jnp.float32)
        + b_ref[...]
    )


def _scale_body(g_ref, e_ref, o_ref):
    o_ref[...] = g_ref[...] * e_ref[...]


_head_out_shapes = (
    jax.ShapeDtypeStruct((R, N, HID), jnp.float32),
    jax.ShapeDtypeStruct((N, 2 * R), jnp.float32),
)
_head_out_specs = (
    pl.BlockSpec((R, _NB, HID), lambda i: (0, i, 0)),
    pl.BlockSpec((_NB, 2 * R), lambda i: (i, 0)),
)
_qk_spec = pl.BlockSpec((1, HID), lambda i: (0, 0))
_pd_specs = [
    pl.BlockSpec((R, _NB, HID), lambda i: (0, i, 0)),
    pl.BlockSpec((R, _NB, 1), lambda i: (0, i, 0)),
]


def _run_head1(x, w, qt, kt):
    return pl.pallas_call(
        _head1,
        grid=(N // _NB,),
        in_specs=[
            pl.BlockSpec((_NB, IN_DIM), lambda i: (i, 0)),
            pl.BlockSpec((R, IN_DIM, HID), lambda i: (0, 0, 0)),
            _qk_spec, _qk_spec,
        ],
        out_specs=_head_out_specs,
        out_shape=_head_out_shapes,
    )(x, w, qt, kt)


def _run_head2(parts, dens, w, qt, kt):
    return pl.pallas_call(
        _head2,
        grid=(N // _NB,),
        in_specs=_pd_specs + [
            pl.BlockSpec((R, HID, HID), lambda i: (0, 0, 0)),
            _qk_spec, _qk_spec,
        ],
        out_specs=_head_out_specs,
        out_shape=_head_out_shapes,
    )(parts, dens, w, qt, kt)


def _run_ubound(s):
    return pl.pallas_call(
        _ubound,
        out_shape=jax.ShapeDtypeStruct((N, 1), jnp.float32),
    )(s)


def _run_decoder(parts, dens, wt, b):
    return pl.pallas_call(
        _decoder,
        grid=(N // _NB,),
        in_specs=_pd_specs + [
            pl.BlockSpec((HID, IN_DIM), lambda i: (0, 0)),
            pl.BlockSpec((1, IN_DIM), lambda i: (0, 0)),
        ],
        out_specs=(
            pl.BlockSpec((_NB, HID), lambda i: (i, 0)),
            pl.BlockSpec((_NB, IN_DIM), lambda i: (i, 0)),
        ),
        out_shape=(
            jax.ShapeDtypeStruct((N, HID), jnp.float32),
            jax.ShapeDtypeStruct((N, IN_DIM), jnp.float32),
        ),
    )(parts, dens, wt, b)


_SCALE_BE = 8192


def _run_scale(gathered, ex_col):
    grid = EPAD // _SCALE_BE
    return pl.pallas_call(
        _scale_body,
        grid=(grid,),
        in_specs=[
            pl.BlockSpec((_SCALE_BE, HID), lambda i: (i, 0)),
            pl.BlockSpec((_SCALE_BE, 1), lambda i: (i, 0)),
        ],
        out_specs=pl.BlockSpec((_SCALE_BE, HID), lambda i: (i, 0)),
        out_shape=jax.ShapeDtypeStruct((EPAD, HID), jnp.float32),
    )(gathered, ex_col)


@functools.partial(
    pl.kernel,
    mesh=_mesh,
    compiler_params=_sc_params,
    out_type=(
        jax.ShapeDtypeStruct((ROWS, 128), jnp.float32),
        jax.ShapeDtypeStruct((2, NPAD), jnp.float32),
    ),
    scratch_types=[
        pltpu.VMEM((SROWS, 128), jnp.float32),
        pltpu.VMEM((SROWS, 128), jnp.float32),
        pltpu.VMEM((UROWS, 128), jnp.float32),
        pltpu.VMEM((ROWS_PER_TILE, 128), jnp.int32),
        pltpu.VMEM((ROWS_PER_TILE, 128), jnp.int32),
        pltpu.VMEM((ROWS_PER_TILE, 128), jnp.int32),
        pltpu.VMEM((ROWS_PER_TILE, 128), jnp.float32),
        pltpu.VMEM((NPAD,), jnp.float32),
        pltpu.VMEM((16, NSLICE), jnp.float32),
        pltpu.VMEM_SHARED((16, NPAD), jnp.float32),
        pltpu.VMEM_SHARED((2 * SROWS + UROWS, 128), jnp.float32),
        pltpu.SemaphoreType.DMA,
    ],
)
def _sc_scores(sq_h, sk_h, u_h, iq_h, ik_h, dd_h,
               ex_h, den_h,
               sq_v, sk_v, u_v, iq_v, ik_v, dd_v, ex_v,
               priv, red_v, stage, tstage, ld_sem):
    c = lax.axis_index("c")
    s = lax.axis_index("s")
    wid = s * 2 + c

    base = wid * ROWS_PER_TILE

    @pl.when(s == 0)
    def _():
        pltpu.sync_copy(sq_h, tstage.at[pl.ds(0, SROWS)])
        pltpu.sync_copy(sk_h, tstage.at[pl.ds(SROWS, SROWS)])
        pltpu.sync_copy(u_h, tstage.at[pl.ds(2 * SROWS, UROWS)])

    zv = jnp.zeros((16,), jnp.float32)

    @pl.loop(0, NPAD // 16)
    def _(v):
        priv[pl.ds(v * 16, 16)] = zv

    lds = [
        pltpu.async_copy(iq_h.at[pl.ds(base, ROWS_PER_TILE)], iq_v, ld_sem),
        pltpu.async_copy(ik_h.at[pl.ds(base, ROWS_PER_TILE)], ik_v, ld_sem),
        pltpu.async_copy(dd_h.at[pl.ds(base, ROWS_PER_TILE)], dd_v, ld_sem),
    ]
    plsc.subcore_barrier()
    pltpu.sync_copy(tstage.at[pl.ds(0, SROWS)], sq_v)
    pltpu.sync_copy(tstage.at[pl.ds(SROWS, SROWS)], sk_v)
    pltpu.sync_copy(tstage.at[pl.ds(2 * SROWS, UROWS)], u_v)
    for h in lds:
        h.wait()

    @pl.loop(0, ROWS_PER_TILE)
    def _(j):
        @pl.loop(0, 8)
        def _(v):
            sl = pl.ds(v * 16, 16)
            dd = dd_v[j, sl]
            iq = iq_v[j, sl]
            ik = ik_v[j, sl]
            a = plsc.load_gather(
                sq_v, [lax.shift_right_logical(iq, 7), iq & 127])
            b = plsc.load_gather(
                sk_v, [lax.shift_right_logical(ik, 7), ik & 127])
            u = plsc.load_gather(
                u_v, [lax.shift_right_logical(dd, 7), dd & 127])
            e = jnp.exp(_lrelu(a + b) - u)
            ex_v[j, sl] = e
            plsc.addupdate_scatter(priv, [dd], e)

    ex_out = pltpu.async_copy(
        ex_v, ex_h.at[pl.ds(base, ROWS_PER_TILE)], ld_sem)
    pltpu.sync_copy(priv, stage.at[s])
    plsc.subcore_barrier()

    pltpu.sync_copy(stage.at[:, pl.ds(s * NSLICE, NSLICE)], red_v)

    @pl.loop(0, NSLICE // 16)
    def _(v):
        sl = pl.ds(v * 16, 16)
        acc = red_v[0, sl]
        for j in range(1, 16):
            acc = acc + red_v[j, sl]
        red_v[0, sl] = acc

    pltpu.sync_copy(red_v.at[0], den_h.at[c, pl.ds(s * NSLICE, NSLICE)])
    ex_out.wait()


@functools.partial(
    pl.kernel,
    mesh=_mesh,
    compiler_params=_sc_lin_params,
    out_type=jax.ShapeDtypeStruct((EPAD, HID), jnp.float32),
    scratch_types=[
        pltpu.VMEM((ROWS_PER_TILE, 128), jnp.int32),
        pltpu.VMEM((512, HID), jnp.float32),
        pltpu.VMEM((512, HID), jnp.float32),
        pltpu.SemaphoreType.DMA,
        pltpu.SemaphoreType.DMA,
        pltpu.SemaphoreType.DMA,
        pltpu.SemaphoreType.DMA,
    ],
)
def _sc_gather(xw_h, ik_h, g_h, ik_v, rows_a, rows_b,
               g_sem_a, g_sem_b, w_sem_a, w_sem_b):
    c = lax.axis_index("c")
    s = lax.axis_index("s")
    wid = s * 2 + c
    base = wid * ROWS_PER_TILE
    pltpu.sync_copy(ik_h.at[pl.ds(base, ROWS_PER_TILE)], ik_v)

    bufs = (rows_a, rows_b)
    g_sems = (g_sem_a, g_sem_b)
    w_sems = (w_sem_a, w_sem_b)
    n_chunks = ROWS_PER_TILE // 4
    w_hs = [None, None]
    for t in range(n_chunks):
        b = t % 2
        if w_hs[b] is not None:
            w_hs[b].wait()
        g_hs = [
            pltpu.async_copy(
                xw_h.at[ik_v.at[t * 4 + j]],
                bufs[b].at[pl.ds(j * 128, 128)],
                g_sems[b])
            for j in range(4)
        ]
        for h in g_hs:
            h.wait()
        w_hs[b] = pltpu.async_copy(
            bufs[b], g_h.at[pl.ds((base + t * 4) * 128, 512)], w_sems[b])
    for h in w_hs:
        if h is not None:
            h.wait()


@functools.partial(
    pl.kernel,
    mesh=_mesh,
    compiler_params=_sc_lin_params,
    out_type=jax.ShapeDtypeStruct((2, NPAD, HID), jnp.float32),
    scratch_types=[
        pltpu.VMEM((ROWS_PER_TILE, 128), jnp.int32),
        pltpu.VMEM((512, HID), jnp.float32),
        pltpu.VMEM((512, HID), jnp.float32),
        pltpu.VMEM_SHARED((NPAD, HID), jnp.float32),
        pltpu.SemaphoreType.DMA,
        pltpu.SemaphoreType.DMA,
        pltpu.SemaphoreType.DMA,
        pltpu.SemaphoreType.DMA,
    ],
)
def _sc_scatter(sc_h, dd_h, z_h, out_h, dd_v, rows_a, rows_b, racc,
                i_sem_a, i_sem_b, s_sem_a, s_sem_b):
    c = lax.axis_index("c")
    s = lax.axis_index("s")
    wid = s * 2 + c

    @pl.when(s == 0)
    def _():
        pltpu.sync_copy(z_h, racc)

    plsc.subcore_barrier()

    base = wid * ROWS_PER_TILE
    pltpu.sync_copy(dd_h.at[pl.ds(base, ROWS_PER_TILE)], dd_v)

    bufs = (rows_a, rows_b)
    i_sems = (i_sem_a, i_sem_b)
    s_sems = (s_sem_a, s_sem_b)
    n_chunks = ROWS_PER_TILE // 4

    def start_in(t):
        b = t % 2
        return pltpu.async_copy(
            sc_h.at[pl.ds((base + t * 4) * 128, 512)], bufs[b], i_sems[b])

    in_hs = [start_in(0), None]
    sc_hs = [[], []]
    for t in range(n_chunks):
        b = t % 2
        if t + 1 < n_chunks:
            nb = (t + 1) % 2
            for h in sc_hs[nb]:
                h.wait()
            sc_hs[nb] = []
            in_hs[nb] = start_in(t + 1)
        in_hs[b].wait()
        sc_hs[b] = [
            pltpu.async_copy(
                bufs[b].at[pl.ds(j * 128, 128)],
                racc.at[dd_v.at[t * 4 + j]],
                s_sems[b], add=True)
            for j in range(4)
        ]
    for hs in sc_hs:
        for h in hs:
            h.wait()

    plsc.subcore_barrier()

    @pl.when(s == 0)
    def _():
        pltpu.sync_copy(racc, out_h.at[c])


def _layer(xw, s, iqp, ikp, ddp, z2):
    u = _run_ubound(s).reshape(N)
    u_pad = jnp.concatenate(
        [u, jnp.full((NPAD - N,), 1000.0, jnp.float32)]).reshape(UROWS, 128)

    def to_2d(col):
        flat = s[:, col:col + R].T.reshape(R * N)
        return jnp.pad(flat, (0, SROWS * 128 - R * N)).reshape(SROWS, 128)

    ex, den = _sc_scores(to_2d(0), to_2d(R), u_pad, iqp, ikp, ddp)
    gathered = _sc_gather(xw.reshape(R * N, HID), ikp)
    scaled = _run_scale(gathered, ex.reshape(EPAD, 1))
    parts = _sc_scatter(scaled, ddp, z2)
    return parts, den.reshape(R, NPAD, 1)


@jax.jit
def kernel(features, edge_index, edge_type, w1, q1, k1, w2, q2, k2,
           dec_w, dec_b):
    src = edge_index[0]
    dst = edge_index[1]
    iq = edge_type * N + dst
    ik = edge_type * N + src

    def pad_to_rows(a, v):
        return jnp.pad(a, (0, EPAD - E), constant_values=v).reshape(ROWS, 128)

    iqp = pad_to_rows(iq, 0)
    ikp = pad_to_rows(ik, 0)
    ddp = pad_to_rows(dst, N)
    z2 = jnp.zeros((NPAD, HID), jnp.float32)

    xw1, s1 = _run_head1(features, w1, q1.T, k1.T)
    parts1, den1 = _layer(xw1, s1, iqp, ikp, ddp, z2)

    xw2, s2 = _run_head2(parts1, den1, w2, q2.T, k2.T)
    parts2, den2 = _layer(xw2, s2, iqp, ikp, ddp, z2)

    h2, h3 = _run_decoder(parts2, den2, dec_w.T, dec_b.reshape(1, IN_DIM))
    return (h2, h3)


# speedup vs baseline: 1.2888x; 1.2732x over previous
import dataclasses
import functools
import jax
import jax.numpy as jnp
from jax import lax
from jax.experimental import pallas as pl
from jax.experimental.pallas import tpu as pltpu
from jax.experimental.pallas import tpu_sc as plsc

N = 10000
E = 320000
R = 2
IN_DIM = 128
HID = 64
NEG = 0.2

NPAD = 10240
NSLICE = NPAD // 16
SROWS = 157
UROWS = NPAD // 128
ROWS = 2560
EPAD = ROWS * 128
NTILES = 32
ROWS_PER_TILE = ROWS // NTILES
SLABS = ROWS_PER_TILE // 8

_mesh = plsc.VectorSubcoreMesh(core_axis_name="c", subcore_axis_name="s")

_sc_params = pltpu.CompilerParams()
if "needs_layout_passes" in pltpu.CompilerParams.__dataclass_fields__:
    _sc_params = dataclasses.replace(_sc_params, needs_layout_passes=False)
_sc_lin_params = _sc_params
if "use_tc_tiling_on_sc" in pltpu.CompilerParams.__dataclass_fields__:
    _sc_lin_params = dataclasses.replace(_sc_params, use_tc_tiling_on_sc=False)


def _lrelu(x):
    return jnp.where(x >= 0.0, x, NEG * x)


def _elu(x):
    return jnp.where(x > 0.0, x, jnp.exp(jnp.minimum(x, 0.0)) - 1.0)


_NB = 1000


def _head_block(x, w_ref, q_ref, k_ref, xw_ref, s_ref):
    cols = [None] * (2 * R)
    for r in range(R):
        xw = jnp.dot(x, w_ref[r], preferred_element_type=jnp.float32)
        xw_ref[r] = xw
        cols[r] = jnp.sum(xw * q_ref[...], axis=1, keepdims=True)
        cols[R + r] = jnp.sum(xw * k_ref[...], axis=1, keepdims=True)
    s_ref[...] = jnp.concatenate(cols, axis=1)


def _head1(x_ref, w_ref, q_ref, k_ref, xw_ref, s_ref):
    _head_block(x_ref[...], w_ref, q_ref, k_ref, xw_ref, s_ref)


def _finalize(p_ref, d_ref):
    d = d_ref[0] + d_ref[1] + 1e-16
    return _elu((p_ref[0] + p_ref[1]) / d)


def _head2(p_ref, d_ref, w_ref, q_ref, k_ref, xw_ref, s_ref):
    _head_block(_finalize(p_ref, d_ref), w_ref, q_ref, k_ref, xw_ref, s_ref)


def _ubound(s_ref, u_ref):
    u = None
    for r in range(R):
        kmax = jnp.max(s_ref[:, R + r:R + r + 1])
        b = _lrelu(s_ref[:, r:r + 1] + kmax)
        u = b if u is None else jnp.maximum(u, b)
    u_ref[...] = u


def _decoder(p_ref, d_ref, w_ref, b_ref, h2_ref, h3_ref):
    h2 = _finalize(p_ref, d_ref)
    h2_ref[...] = h2
    h3_ref[...] = (
        jnp.dot(h2, w_ref[...], preferred_element_type=jnp.float32)
        + b_ref[...]
    )


def _scale_body(g_ref, e_ref, o_ref):
    o_ref[...] = g_ref[...] * e_ref[...]


_head_out_shapes = (
    jax.ShapeDtypeStruct((R, N, HID), jnp.float32),
    jax.ShapeDtypeStruct((N, 2 * R), jnp.float32),
)
_head_out_specs = (
    pl.BlockSpec((R, _NB, HID), lambda i: (0, i, 0)),
    pl.BlockSpec((_NB, 2 * R), lambda i: (i, 0)),
)
_qk_spec = pl.BlockSpec((1, HID), lambda i: (0, 0))
_pd_specs = [
    pl.BlockSpec((R, _NB, HID), lambda i: (0, i, 0)),
    pl.BlockSpec((R, _NB, 1), lambda i: (0, i, 0)),
]


def _run_head1(x, w, qt, kt):
    return pl.pallas_call(
        _head1,
        grid=(N // _NB,),
        in_specs=[
            pl.BlockSpec((_NB, IN_DIM), lambda i: (i, 0)),
            pl.BlockSpec((R, IN_DIM, HID), lambda i: (0, 0, 0)),
            _qk_spec, _qk_spec,
        ],
        out_specs=_head_out_specs,
        out_shape=_head_out_shapes,
    )(x, w, qt, kt)


def _run_head2(parts, dens, w, qt, kt):
    return pl.pallas_call(
        _head2,
        grid=(N // _NB,),
        in_specs=_pd_specs + [
            pl.BlockSpec((R, HID, HID), lambda i: (0, 0, 0)),
            _qk_spec, _qk_spec,
        ],
        out_specs=_head_out_specs,
        out_shape=_head_out_shapes,
    )(parts, dens, w, qt, kt)


def _run_ubound(s):
    return pl.pallas_call(
        _ubound,
        out_shape=jax.ShapeDtypeStruct((N, 1), jnp.float32),
    )(s)


def _run_decoder(parts, dens, wt, b):
    return pl.pallas_call(
        _decoder,
        grid=(N // _NB,),
        in_specs=_pd_specs + [
            pl.BlockSpec((HID, IN_DIM), lambda i: (0, 0)),
            pl.BlockSpec((1, IN_DIM), lambda i: (0, 0)),
        ],
        out_specs=(
            pl.BlockSpec((_NB, HID), lambda i: (i, 0)),
            pl.BlockSpec((_NB, IN_DIM), lambda i: (i, 0)),
        ),
        out_shape=(
            jax.ShapeDtypeStruct((N, HID), jnp.float32),
            jax.ShapeDtypeStruct((N, IN_DIM), jnp.float32),
        ),
    )(parts, dens, wt, b)


_SCALE_BE = 8192


def _run_scale(gathered, ex_col):
    grid = EPAD // _SCALE_BE
    return pl.pallas_call(
        _scale_body,
        grid=(grid,),
        in_specs=[
            pl.BlockSpec((_SCALE_BE, HID), lambda i: (i, 0)),
            pl.BlockSpec((_SCALE_BE, 1), lambda i: (i, 0)),
        ],
        out_specs=pl.BlockSpec((_SCALE_BE, HID), lambda i: (i, 0)),
        out_shape=jax.ShapeDtypeStruct((EPAD, HID), jnp.float32),
    )(gathered, ex_col)


@functools.partial(
    pl.kernel,
    mesh=_mesh,
    compiler_params=_sc_params,
    out_type=(
        jax.ShapeDtypeStruct((ROWS, 128), jnp.float32),
        jax.ShapeDtypeStruct((2, NPAD), jnp.float32),
    ),
    scratch_types=[
        pltpu.VMEM((SROWS, 128), jnp.float32),
        pltpu.VMEM((SROWS, 128), jnp.float32),
        pltpu.VMEM((UROWS, 128), jnp.float32),
        pltpu.VMEM((ROWS_PER_TILE, 128), jnp.int32),
        pltpu.VMEM((ROWS_PER_TILE, 128), jnp.int32),
        pltpu.VMEM((ROWS_PER_TILE, 128), jnp.int32),
        pltpu.VMEM((ROWS_PER_TILE, 128), jnp.float32),
        pltpu.VMEM((NPAD,), jnp.float32),
        pltpu.VMEM((16, NSLICE), jnp.float32),
        pltpu.VMEM_SHARED((16, NPAD), jnp.float32),
        pltpu.VMEM_SHARED((2 * SROWS + UROWS, 128), jnp.float32),
        pltpu.SemaphoreType.DMA,
    ],
)
def _sc_scores(sq_h, sk_h, u_h, iq_h, ik_h, dd_h,
               ex_h, den_h,
               sq_v, sk_v, u_v, iq_v, ik_v, dd_v, ex_v,
               priv, red_v, stage, tstage, ld_sem):
    c = lax.axis_index("c")
    s = lax.axis_index("s")
    wid = s * 2 + c

    base = wid * ROWS_PER_TILE

    @pl.when(s == 0)
    def _():
        pltpu.sync_copy(sq_h, tstage.at[pl.ds(0, SROWS)])
        pltpu.sync_copy(sk_h, tstage.at[pl.ds(SROWS, SROWS)])
        pltpu.sync_copy(u_h, tstage.at[pl.ds(2 * SROWS, UROWS)])

    zv = jnp.zeros((16,), jnp.float32)

    @pl.loop(0, NPAD // 16)
    def _(v):
        priv[pl.ds(v * 16, 16)] = zv

    lds = [
        pltpu.async_copy(iq_h.at[pl.ds(base, ROWS_PER_TILE)], iq_v, ld_sem),
        pltpu.async_copy(ik_h.at[pl.ds(base, ROWS_PER_TILE)], ik_v, ld_sem),
        pltpu.async_copy(dd_h.at[pl.ds(base, ROWS_PER_TILE)], dd_v, ld_sem),
    ]
    plsc.subcore_barrier()
    pltpu.sync_copy(tstage.at[pl.ds(0, SROWS)], sq_v)
    pltpu.sync_copy(tstage.at[pl.ds(SROWS, SROWS)], sk_v)
    pltpu.sync_copy(tstage.at[pl.ds(2 * SROWS, UROWS)], u_v)
    for h in lds:
        h.wait()

    @pl.loop(0, ROWS_PER_TILE)
    def _(j):
        @pl.loop(0, 8)
        def _(v):
            sl = pl.ds(v * 16, 16)
            dd = dd_v[j, sl]
            iq = iq_v[j, sl]
            ik = ik_v[j, sl]
            a = plsc.load_gather(
                sq_v, [lax.shift_right_logical(iq, 7), iq & 127])
            b = plsc.load_gather(
                sk_v, [lax.shift_right_logical(ik, 7), ik & 127])
            u = plsc.load_gather(
                u_v, [lax.shift_right_logical(dd, 7), dd & 127])
            e = jnp.exp(_lrelu(a + b) - u)
            ex_v[j, sl] = e
            plsc.addupdate_scatter(priv, [dd], e)

    ex_out = pltpu.async_copy(
        ex_v, ex_h.at[pl.ds(base, ROWS_PER_TILE)], ld_sem)
    pltpu.sync_copy(priv, stage.at[s])
    plsc.subcore_barrier()

    pltpu.sync_copy(stage.at[:, pl.ds(s * NSLICE, NSLICE)], red_v)

    @pl.loop(0, NSLICE // 16)
    def _(v):
        sl = pl.ds(v * 16, 16)
        acc = red_v[0, sl]
        for j in range(1, 16):
            acc = acc + red_v[j, sl]
        red_v[0, sl] = acc

    pltpu.sync_copy(red_v.at[0], den_h.at[c, pl.ds(s * NSLICE, NSLICE)])
    ex_out.wait()


@functools.partial(
    pl.kernel,
    mesh=_mesh,
    compiler_params=_sc_lin_params,
    out_type=jax.ShapeDtypeStruct((EPAD, HID), jnp.float32),
    scratch_types=[
        pltpu.VMEM((ROWS_PER_TILE, 128), jnp.int32),
        pltpu.VMEM((512, HID), jnp.float32),
        pltpu.VMEM((512, HID), jnp.float32),
        pltpu.SemaphoreType.DMA,
        pltpu.SemaphoreType.DMA,
        pltpu.SemaphoreType.DMA,
        pltpu.SemaphoreType.DMA,
    ],
)
def _sc_gather(xw_h, ik_h, g_h, ik_v, rows_a, rows_b,
               g_sem_a, g_sem_b, w_sem_a, w_sem_b):
    c = lax.axis_index("c")
    s = lax.axis_index("s")
    wid = s * 2 + c
    base = wid * ROWS_PER_TILE
    pltpu.sync_copy(ik_h.at[pl.ds(base, ROWS_PER_TILE)], ik_v)

    bufs = (rows_a, rows_b)
    g_sems = (g_sem_a, g_sem_b)
    w_sems = (w_sem_a, w_sem_b)
    n_chunks = ROWS_PER_TILE // 4
    w_hs = [None, None]
    for t in range(n_chunks):
        b = t % 2
        if w_hs[b] is not None:
            w_hs[b].wait()
        g_hs = [
            pltpu.async_copy(
                xw_h.at[ik_v.at[t * 4 + j]],
                bufs[b].at[pl.ds(j * 128, 128)],
                g_sems[b])
            for j in range(4)
        ]
        for h in g_hs:
            h.wait()
        w_hs[b] = pltpu.async_copy(
            bufs[b], g_h.at[pl.ds((base + t * 4) * 128, 512)], w_sems[b])
    for h in w_hs:
        if h is not None:
            h.wait()


@functools.partial(
    pl.kernel,
    mesh=_mesh,
    compiler_params=_sc_lin_params,
    out_type=jax.ShapeDtypeStruct((2, NPAD, HID), jnp.float32),
    scratch_types=[
        pltpu.VMEM((ROWS_PER_TILE, 128), jnp.int32),
        pltpu.VMEM((512, HID), jnp.float32),
        pltpu.VMEM((512, HID), jnp.float32),
        pltpu.VMEM_SHARED((NPAD, HID), jnp.float32),
        pltpu.SemaphoreType.DMA,
        pltpu.SemaphoreType.DMA,
        pltpu.SemaphoreType.DMA,
        pltpu.SemaphoreType.DMA,
    ],
)
def _sc_scatter(sc_h, dd_h, z_h, out_h, dd_v, rows_a, rows_b, racc,
                i_sem_a, i_sem_b, s_sem_a, s_sem_b):
    c = lax.axis_index("c")
    s = lax.axis_index("s")
    wid = s * 2 + c

    @pl.when(s == 0)
    def _():
        pltpu.sync_copy(z_h, racc)

    plsc.subcore_barrier()

    base = wid * ROWS_PER_TILE
    pltpu.sync_copy(dd_h.at[pl.ds(base, ROWS_PER_TILE)], dd_v)

    bufs = (rows_a, rows_b)
    i_sems = (i_sem_a, i_sem_b)
    s_sems = (s_sem_a, s_sem_b)
    n_chunks = ROWS_PER_TILE // 4

    def start_in(t):
        b = t % 2
        return pltpu.async_copy(
            sc_h.at[pl.ds((base + t * 4) * 128, 512)], bufs[b], i_sems[b])

    in_hs = [start_in(0), None]
    sc_hs = [[], []]
    for t in range(n_chunks):
        b = t % 2
        if t + 1 < n_chunks:
            nb = (t + 1) % 2
            for h in sc_hs[nb]:
                h.wait()
            sc_hs[nb] = []
            in_hs[nb] = start_in(t + 1)
        in_hs[b].wait()
        sc_hs[b] = [
            pltpu.async_copy(
                bufs[b].at[pl.ds(j * 128, 128)],
                racc.at[dd_v.at[t * 4 + j]],
                s_sems[b], add=True)
            for j in range(4)
        ]
    for hs in sc_hs:
        for h in hs:
            h.wait()

    plsc.subcore_barrier()

    @pl.when(s == 0)
    def _():
        pltpu.sync_copy(racc, out_h.at[c])


def _layer(xw, s, iqp, ikp, ddp, z2):
    u = _run_ubound(s).reshape(N)
    u_pad = jnp.concatenate(
        [u, jnp.full((NPAD - N,), 1000.0, jnp.float32)]).reshape(UROWS, 128)

    def to_2d(col):
        flat = s[:, col:col + R].T.reshape(R * N)
        return jnp.pad(flat, (0, SROWS * 128 - R * N)).reshape(SROWS, 128)

    ex, den = _sc_scores(to_2d(0), to_2d(R), u_pad, iqp, ikp, ddp)
    gathered = _sc_gather(xw.reshape(R * N, HID), ikp)
    scaled = _run_scale(gathered, ex.reshape(EPAD, 1))
    parts = _sc_scatter(scaled, ddp, z2)
    return parts, den.reshape(R, NPAD, 1)


@jax.jit
def kernel(features, edge_index, edge_type, w1, q1, k1, w2, q2, k2,
           dec_w, dec_b):
    src = edge_index[0]
    dst = edge_index[1]
    iq = edge_type * N + dst
    ik = edge_type * N + src

    pad_iota = jax.lax.iota(jnp.int32, EPAD - E)

    def pad_to_rows(a, pad_vals):
        return jnp.concatenate([a, pad_vals]).reshape(ROWS, 128)

    iqp = pad_to_rows(iq, pad_iota % 128)
    ikp = pad_to_rows(ik, pad_iota % 128)
    ddp = pad_to_rows(dst, N + pad_iota % (NPAD - N))
    z2 = jnp.zeros((NPAD, HID), jnp.float32)

    xw1, s1 = _run_head1(features, w1, q1.T, k1.T)
    parts1, den1 = _layer(xw1, s1, iqp, ikp, ddp, z2)

    xw2, s2 = _run_head2(parts1, den1, w2, q2.T, k2.T)
    parts2, den2 = _layer(xw2, s2, iqp, ikp, ddp, z2)

    h2, h3 = _run_decoder(parts2, den2, dec_w.T, dec_b.reshape(1, IN_DIM))
    return (h2, h3)


# speedup vs baseline: 1.3846x; 1.0743x over previous
import dataclasses
import functools
import jax
import jax.numpy as jnp
from jax import lax
from jax.experimental import pallas as pl
from jax.experimental.pallas import tpu as pltpu
from jax.experimental.pallas import tpu_sc as plsc

N = 10000
E = 320000
R = 2
IN_DIM = 128
HID = 64
NEG = 0.2

NPAD = 10240
NSLICE = NPAD // 16
SROWS = 157
UROWS = NPAD // 128
ROWS = 2560
EPAD = ROWS * 128
NTILES = 32
ROWS_PER_TILE = ROWS // NTILES
SLABS = ROWS_PER_TILE // 8

_mesh = plsc.VectorSubcoreMesh(core_axis_name="c", subcore_axis_name="s")

_sc_params = pltpu.CompilerParams()
if "needs_layout_passes" in pltpu.CompilerParams.__dataclass_fields__:
    _sc_params = dataclasses.replace(_sc_params, needs_layout_passes=False)
_sc_lin_params = _sc_params
if "use_tc_tiling_on_sc" in pltpu.CompilerParams.__dataclass_fields__:
    _sc_lin_params = dataclasses.replace(_sc_params, use_tc_tiling_on_sc=False)


def _lrelu(x):
    return jnp.where(x >= 0.0, x, NEG * x)


def _elu(x):
    return jnp.where(x > 0.0, x, jnp.exp(jnp.minimum(x, 0.0)) - 1.0)


_NB = 1000


def _head_block(x, w_ref, q_ref, k_ref, xw_ref, s_ref):
    cols = [None] * (2 * R)
    xws = []
    for r in range(R):
        xw = jnp.dot(x, w_ref[r], preferred_element_type=jnp.float32)
        xws.append(xw)
        cols[r] = jnp.sum(xw * q_ref[...], axis=1, keepdims=True)
        cols[R + r] = jnp.sum(xw * k_ref[...], axis=1, keepdims=True)
    xw_ref[...] = jnp.concatenate(xws, axis=1)
    s_ref[...] = jnp.concatenate(cols, axis=1)


def _head1(x_ref, w_ref, q_ref, k_ref, xw_ref, s_ref):
    _head_block(x_ref[...], w_ref, q_ref, k_ref, xw_ref, s_ref)


def _finalize(p_ref, d_ref):
    d = d_ref[0] + d_ref[1] + 1e-16
    return _elu((p_ref[0] + p_ref[1]) / d)


def _head2(p_ref, d_ref, w_ref, q_ref, k_ref, xw_ref, s_ref):
    _head_block(_finalize(p_ref, d_ref), w_ref, q_ref, k_ref, xw_ref, s_ref)


def _ubound(s_ref, u_ref):
    u = None
    for r in range(R):
        kmax = jnp.max(s_ref[:, R + r:R + r + 1])
        b = _lrelu(s_ref[:, r:r + 1] + kmax)
        u = b if u is None else jnp.maximum(u, b)
    u_ref[...] = u


def _decoder(p_ref, d_ref, w_ref, b_ref, h2_ref, h3_ref):
    h2 = _finalize(p_ref, d_ref)
    h2_ref[...] = h2
    h3_ref[...] = (
        jnp.dot(h2, w_ref[...], preferred_element_type=jnp.float32)
        + b_ref[...]
    )


def _scale_body(g_ref, e_ref, o_ref):
    g = g_ref[...]
    o_ref[...] = (g[:, :HID] * e_ref[:, 0:1] + g[:, HID:] * e_ref[:, 1:2])


_head_out_shapes = (
    jax.ShapeDtypeStruct((N, R * HID), jnp.float32),
    jax.ShapeDtypeStruct((N, 2 * R), jnp.float32),
)
_head_out_specs = (
    pl.BlockSpec((_NB, R * HID), lambda i: (i, 0)),
    pl.BlockSpec((_NB, 2 * R), lambda i: (i, 0)),
)
_qk_spec = pl.BlockSpec((1, HID), lambda i: (0, 0))
_pd_specs = [
    pl.BlockSpec((R, _NB, HID), lambda i: (0, i, 0)),
    pl.BlockSpec((R, _NB, 1), lambda i: (0, i, 0)),
]


def _run_head1(x, w, qt, kt):
    return pl.pallas_call(
        _head1,
        grid=(N // _NB,),
        in_specs=[
            pl.BlockSpec((_NB, IN_DIM), lambda i: (i, 0)),
            pl.BlockSpec((R, IN_DIM, HID), lambda i: (0, 0, 0)),
            _qk_spec, _qk_spec,
        ],
        out_specs=_head_out_specs,
        out_shape=_head_out_shapes,
    )(x, w, qt, kt)


def _run_head2(parts, dens, w, qt, kt):
    return pl.pallas_call(
        _head2,
        grid=(N // _NB,),
        in_specs=_pd_specs + [
            pl.BlockSpec((R, HID, HID), lambda i: (0, 0, 0)),
            _qk_spec, _qk_spec,
        ],
        out_specs=_head_out_specs,
        out_shape=_head_out_shapes,
    )(parts, dens, w, qt, kt)


def _run_ubound(s):
    return pl.pallas_call(
        _ubound,
        out_shape=jax.ShapeDtypeStruct((N, 1), jnp.float32),
    )(s)


def _run_decoder(parts, dens, wt, b):
    return pl.pallas_call(
        _decoder,
        grid=(N // _NB,),
        in_specs=_pd_specs + [
            pl.BlockSpec((HID, IN_DIM), lambda i: (0, 0)),
            pl.BlockSpec((1, IN_DIM), lambda i: (0, 0)),
        ],
        out_specs=(
            pl.BlockSpec((_NB, HID), lambda i: (i, 0)),
            pl.BlockSpec((_NB, IN_DIM), lambda i: (i, 0)),
        ),
        out_shape=(
            jax.ShapeDtypeStruct((N, HID), jnp.float32),
            jax.ShapeDtypeStruct((N, IN_DIM), jnp.float32),
        ),
    )(parts, dens, wt, b)


_SCALE_BE = 8192


def _run_scale(gathered, ex_cols):
    grid = EPAD // _SCALE_BE
    return pl.pallas_call(
        _scale_body,
        grid=(grid,),
        in_specs=[
            pl.BlockSpec((_SCALE_BE, R * HID), lambda i: (i, 0)),
            pl.BlockSpec((_SCALE_BE, 2), lambda i: (i, 0)),
        ],
        out_specs=pl.BlockSpec((_SCALE_BE, HID), lambda i: (i, 0)),
        out_shape=jax.ShapeDtypeStruct((EPAD, HID), jnp.float32),
    )(gathered, ex_cols)


@functools.partial(
    pl.kernel,
    mesh=_mesh,
    compiler_params=_sc_params,
    out_type=(
        jax.ShapeDtypeStruct((ROWS, 128), jnp.float32),
        jax.ShapeDtypeStruct((2, NPAD), jnp.float32),
    ),
    scratch_types=[
        pltpu.VMEM((SROWS, 128), jnp.float32),
        pltpu.VMEM((SROWS, 128), jnp.float32),
        pltpu.VMEM((UROWS, 128), jnp.float32),
        pltpu.VMEM((ROWS_PER_TILE, 128), jnp.int32),
        pltpu.VMEM((ROWS_PER_TILE, 128), jnp.int32),
        pltpu.VMEM((ROWS_PER_TILE, 128), jnp.int32),
        pltpu.VMEM((ROWS_PER_TILE, 128), jnp.float32),
        pltpu.VMEM((NPAD,), jnp.float32),
        pltpu.VMEM((16, NSLICE), jnp.float32),
        pltpu.VMEM_SHARED((16, NPAD), jnp.float32),
        pltpu.VMEM_SHARED((2 * SROWS + UROWS, 128), jnp.float32),
        pltpu.SemaphoreType.DMA,
    ],
)
def _sc_scores(sq_h, sk_h, u_h, iq_h, ik_h, dd_h,
               ex_h, den_h,
               sq_v, sk_v, u_v, iq_v, ik_v, dd_v, ex_v,
               priv, red_v, stage, tstage, ld_sem):
    c = lax.axis_index("c")
    s = lax.axis_index("s")
    wid = s * 2 + c

    base = wid * ROWS_PER_TILE

    @pl.when(s == 0)
    def _():
        pltpu.sync_copy(sq_h, tstage.at[pl.ds(0, SROWS)])
        pltpu.sync_copy(sk_h, tstage.at[pl.ds(SROWS, SROWS)])
        pltpu.sync_copy(u_h, tstage.at[pl.ds(2 * SROWS, UROWS)])

    zv = jnp.zeros((16,), jnp.float32)

    @pl.loop(0, NPAD // 16)
    def _(v):
        priv[pl.ds(v * 16, 16)] = zv

    lds = [
        pltpu.async_copy(iq_h.at[pl.ds(base, ROWS_PER_TILE)], iq_v, ld_sem),
        pltpu.async_copy(ik_h.at[pl.ds(base, ROWS_PER_TILE)], ik_v, ld_sem),
        pltpu.async_copy(dd_h.at[pl.ds(base, ROWS_PER_TILE)], dd_v, ld_sem),
    ]
    plsc.subcore_barrier()
    pltpu.sync_copy(tstage.at[pl.ds(0, SROWS)], sq_v)
    pltpu.sync_copy(tstage.at[pl.ds(SROWS, SROWS)], sk_v)
    pltpu.sync_copy(tstage.at[pl.ds(2 * SROWS, UROWS)], u_v)
    for h in lds:
        h.wait()

    @pl.loop(0, ROWS_PER_TILE)
    def _(j):
        @pl.loop(0, 8)
        def _(v):
            sl = pl.ds(v * 16, 16)
            dd = dd_v[j, sl]
            iq = iq_v[j, sl]
            ik = ik_v[j, sl]
            a = plsc.load_gather(
                sq_v, [lax.shift_right_logical(iq, 7), iq & 127])
            b = plsc.load_gather(
                sk_v, [lax.shift_right_logical(ik, 7), ik & 127])
            u = plsc.load_gather(
                u_v, [lax.shift_right_logical(dd, 7), dd & 127])
            e = jnp.exp(_lrelu(a + b) - u)
            ex_v[j, sl] = e
            plsc.addupdate_scatter(priv, [dd], e)

    ex_out = pltpu.async_copy(
        ex_v, ex_h.at[pl.ds(base, ROWS_PER_TILE)], ld_sem)
    pltpu.sync_copy(priv, stage.at[s])
    plsc.subcore_barrier()

    pltpu.sync_copy(stage.at[:, pl.ds(s * NSLICE, NSLICE)], red_v)

    @pl.loop(0, NSLICE // 16)
    def _(v):
        sl = pl.ds(v * 16, 16)
        acc = red_v[0, sl]
        for j in range(1, 16):
            acc = acc + red_v[j, sl]
        red_v[0, sl] = acc

    pltpu.sync_copy(red_v.at[0], den_h.at[c, pl.ds(s * NSLICE, NSLICE)])
    ex_out.wait()


@functools.partial(
    pl.kernel,
    mesh=_mesh,
    compiler_params=_sc_params,
    out_type=jax.ShapeDtypeStruct((EPAD, R * HID), jnp.float32),
    scratch_types=[
        pltpu.VMEM((ROWS_PER_TILE, 128), jnp.int32),
        pltpu.VMEM((256, R * HID), jnp.float32),
        pltpu.VMEM((256, R * HID), jnp.float32),
        pltpu.SemaphoreType.DMA,
        pltpu.SemaphoreType.DMA,
        pltpu.SemaphoreType.DMA,
        pltpu.SemaphoreType.DMA,
    ],
)
def _sc_gather(xw_h, ik_h, g_h, ik_v, rows_a, rows_b,
               g_sem_a, g_sem_b, w_sem_a, w_sem_b):
    c = lax.axis_index("c")
    s = lax.axis_index("s")
    wid = s * 2 + c
    base = wid * ROWS_PER_TILE
    pltpu.sync_copy(ik_h.at[pl.ds(base, ROWS_PER_TILE)], ik_v)

    bufs = (rows_a, rows_b)
    g_sems = (g_sem_a, g_sem_b)
    w_sems = (w_sem_a, w_sem_b)
    n_chunks = ROWS_PER_TILE // 2
    w_hs = [None, None]
    for t in range(n_chunks):
        b = t % 2
        if w_hs[b] is not None:
            w_hs[b].wait()
        g_hs = [
            pltpu.async_copy(
                xw_h.at[ik_v.at[t * 2 + j]],
                bufs[b].at[pl.ds(j * 128, 128)],
                g_sems[b])
            for j in range(2)
        ]
        for h in g_hs:
            h.wait()
        w_hs[b] = pltpu.async_copy(
            bufs[b], g_h.at[pl.ds((base + t * 2) * 128, 256)], w_sems[b])
    for h in w_hs:
        if h is not None:
            h.wait()


@functools.partial(
    pl.kernel,
    mesh=_mesh,
    compiler_params=_sc_lin_params,
    out_type=jax.ShapeDtypeStruct((2, NPAD, HID), jnp.float32),
    scratch_types=[
        pltpu.VMEM((ROWS_PER_TILE, 128), jnp.int32),
        pltpu.VMEM((512, HID), jnp.float32),
        pltpu.VMEM((512, HID), jnp.float32),
        pltpu.VMEM_SHARED((NPAD, HID), jnp.float32),
        pltpu.SemaphoreType.DMA,
        pltpu.SemaphoreType.DMA,
        pltpu.SemaphoreType.DMA,
        pltpu.SemaphoreType.DMA,
    ],
)
def _sc_scatter(sc_h, dd_h, z_h, out_h, dd_v, rows_a, rows_b, racc,
                i_sem_a, i_sem_b, s_sem_a, s_sem_b):
    c = lax.axis_index("c")
    s = lax.axis_index("s")
    wid = s * 2 + c

    @pl.when(s == 0)
    def _():
        pltpu.sync_copy(z_h, racc)

    plsc.subcore_barrier()

    base = wid * ROWS_PER_TILE
    pltpu.sync_copy(dd_h.at[pl.ds(base, ROWS_PER_TILE)], dd_v)

    bufs = (rows_a, rows_b)
    i_sems = (i_sem_a, i_sem_b)
    s_sems = (s_sem_a, s_sem_b)
    n_chunks = ROWS_PER_TILE // 4

    def start_in(t):
        b = t % 2
        return pltpu.async_copy(
            sc_h.at[pl.ds((base + t * 4) * 128, 512)], bufs[b], i_sems[b])

    in_hs = [start_in(0), None]
    sc_hs = [[], []]
    for t in range(n_chunks):
        b = t % 2
        if t + 1 < n_chunks:
            nb = (t + 1) % 2
            for h in sc_hs[nb]:
                h.wait()
            sc_hs[nb] = []
            in_hs[nb] = start_in(t + 1)
        in_hs[b].wait()
        sc_hs[b] = [
            pltpu.async_copy(
                bufs[b].at[pl.ds(j * 128, 128)],
                racc.at[dd_v.at[t * 4 + j]],
                s_sems[b], add=True)
            for j in range(4)
        ]
    for hs in sc_hs:
        for h in hs:
            h.wait()

    plsc.subcore_barrier()

    @pl.when(s == 0)
    def _():
        pltpu.sync_copy(racc, out_h.at[c])


def _layer(xw, s, iqp, ikp, srcp, ddp, tp, z2):
    u = _run_ubound(s).reshape(N)
    u_pad = jnp.concatenate(
        [u, jnp.full((NPAD - N,), 1000.0, jnp.float32)]).reshape(UROWS, 128)

    def to_2d(col):
        flat = s[:, col:col + R].T.reshape(R * N)
        return jnp.pad(flat, (0, SROWS * 128 - R * N)).reshape(SROWS, 128)

    ex, den = _sc_scores(to_2d(0), to_2d(R), u_pad, iqp, ikp, ddp)
    gathered = _sc_gather(xw, srcp)
    exc = ex.reshape(EPAD, 1)
    exc = jnp.concatenate([exc * (1.0 - tp), exc * tp], axis=1)
    scaled = _run_scale(gathered, exc)
    parts = _sc_scatter(scaled, ddp, z2)
    return parts, den.reshape(R, NPAD, 1)


@jax.jit
def kernel(features, edge_index, edge_type, w1, q1, k1, w2, q2, k2,
           dec_w, dec_b):
    src = edge_index[0]
    dst = edge_index[1]
    iq = edge_type * N + dst
    ik = edge_type * N + src

    pad_iota = jax.lax.iota(jnp.int32, EPAD - E)

    def pad_to_rows(a, pad_vals):
        return jnp.concatenate([a, pad_vals]).reshape(ROWS, 128)

    iqp = pad_to_rows(iq, pad_iota % 128)
    ikp = pad_to_rows(ik, pad_iota % 128)
    srcp = pad_to_rows(src, pad_iota % N)
    ddp = pad_to_rows(dst, N + pad_iota % (NPAD - N))
    tp = jnp.pad(edge_type.astype(jnp.float32),
                 (0, EPAD - E)).reshape(EPAD, 1)
    z2 = jnp.zeros((NPAD, HID), jnp.float32)

    xw1, s1 = _run_head1(features, w1, q1.T, k1.T)
    parts1, den1 = _layer(xw1, s1, iqp, ikp, srcp, ddp, tp, z2)

    xw2, s2 = _run_head2(parts1, den1, w2, q2.T, k2.T)
    parts2, den2 = _layer(xw2, s2, iqp, ikp, srcp, ddp, tp, z2)

    h2, h3 = _run_decoder(parts2, den2, dec_w.T, dec_b.reshape(1, IN_DIM))
    return (h2, h3)


# speedup vs baseline: 1.3931x; 1.0061x over previous
import dataclasses
import functools
import jax
import jax.numpy as jnp
from jax import lax
from jax.experimental import pallas as pl
from jax.experimental.pallas import tpu as pltpu
from jax.experimental.pallas import tpu_sc as plsc

N = 10000
E = 320000
R = 2
IN_DIM = 128
HID = 64
NEG = 0.2

NPAD = 10240
NSLICE = NPAD // 16
SROWS = 157
UROWS = NPAD // 128
ROWS = 2560
EPAD = ROWS * 128
NTILES = 32
ROWS_PER_TILE = ROWS // NTILES
SLABS = ROWS_PER_TILE // 8

_mesh = plsc.VectorSubcoreMesh(core_axis_name="c", subcore_axis_name="s")

_sc_params = pltpu.CompilerParams()
if "needs_layout_passes" in pltpu.CompilerParams.__dataclass_fields__:
    _sc_params = dataclasses.replace(_sc_params, needs_layout_passes=False)
_sc_lin_params = _sc_params
if "use_tc_tiling_on_sc" in pltpu.CompilerParams.__dataclass_fields__:
    _sc_lin_params = dataclasses.replace(_sc_params, use_tc_tiling_on_sc=False)


def _lrelu(x):
    return jnp.where(x >= 0.0, x, NEG * x)


def _elu(x):
    return jnp.where(x > 0.0, x, jnp.exp(jnp.minimum(x, 0.0)) - 1.0)


_NB = 1000


def _head_block(x, w_ref, q_ref, k_ref, xw_ref, s_ref):
    cols = [None] * (2 * R)
    xws = []
    for r in range(R):
        xw = jnp.dot(x, w_ref[r], preferred_element_type=jnp.float32)
        xws.append(xw)
        cols[r] = jnp.sum(xw * q_ref[...], axis=1, keepdims=True)
        cols[R + r] = jnp.sum(xw * k_ref[...], axis=1, keepdims=True)
    xw_ref[...] = jnp.concatenate(xws, axis=1)
    s_ref[...] = jnp.concatenate(cols, axis=1)


def _head1(x_ref, w_ref, q_ref, k_ref, xw_ref, s_ref):
    _head_block(x_ref[...], w_ref, q_ref, k_ref, xw_ref, s_ref)


def _finalize(p_ref, d_ref):
    d = d_ref[0] + d_ref[1] + 1e-16
    return _elu((p_ref[0] + p_ref[1]) / d)


def _head2(p_ref, d_ref, w_ref, q_ref, k_ref, xw_ref, s_ref):
    _head_block(_finalize(p_ref, d_ref), w_ref, q_ref, k_ref, xw_ref, s_ref)


def _ubound(s_ref, u_ref):
    u = None
    for r in range(R):
        kmax = jnp.max(s_ref[:, R + r:R + r + 1])
        b = _lrelu(s_ref[:, r:r + 1] + kmax)
        u = b if u is None else jnp.maximum(u, b)
    u_ref[...] = u


def _decoder(p_ref, d_ref, w_ref, b_ref, h2_ref, h3_ref):
    h2 = _finalize(p_ref, d_ref)
    h2_ref[...] = h2
    h3_ref[...] = (
        jnp.dot(h2, w_ref[...], preferred_element_type=jnp.float32)
        + b_ref[...]
    )


def _scale_body(g_ref, e_ref, o_ref):
    g = g_ref[...]
    o_ref[...] = (g[:, :HID] * e_ref[:, 0:1] + g[:, HID:] * e_ref[:, 1:2])


_head_out_shapes = (
    jax.ShapeDtypeStruct((N, R * HID), jnp.float32),
    jax.ShapeDtypeStruct((N, 2 * R), jnp.float32),
)
_head_out_specs = (
    pl.BlockSpec((_NB, R * HID), lambda i: (i, 0)),
    pl.BlockSpec((_NB, 2 * R), lambda i: (i, 0)),
)
_qk_spec = pl.BlockSpec((1, HID), lambda i: (0, 0))
_pd_specs = [
    pl.BlockSpec((R, _NB, HID), lambda i: (0, i, 0)),
    pl.BlockSpec((R, _NB, 1), lambda i: (0, i, 0)),
]


def _run_head1(x, w, qt, kt):
    return pl.pallas_call(
        _head1,
        grid=(N // _NB,),
        in_specs=[
            pl.BlockSpec((_NB, IN_DIM), lambda i: (i, 0)),
            pl.BlockSpec((R, IN_DIM, HID), lambda i: (0, 0, 0)),
            _qk_spec, _qk_spec,
        ],
        out_specs=_head_out_specs,
        out_shape=_head_out_shapes,
    )(x, w, qt, kt)


def _run_head2(parts, dens, w, qt, kt):
    return pl.pallas_call(
        _head2,
        grid=(N // _NB,),
        in_specs=_pd_specs + [
            pl.BlockSpec((R, HID, HID), lambda i: (0, 0, 0)),
            _qk_spec, _qk_spec,
        ],
        out_specs=_head_out_specs,
        out_shape=_head_out_shapes,
    )(parts, dens, w, qt, kt)


def _run_ubound(s):
    return pl.pallas_call(
        _ubound,
        out_shape=jax.ShapeDtypeStruct((N, 1), jnp.float32),
    )(s)


def _run_decoder(parts, dens, wt, b):
    return pl.pallas_call(
        _decoder,
        grid=(N // _NB,),
        in_specs=_pd_specs + [
            pl.BlockSpec((HID, IN_DIM), lambda i: (0, 0)),
            pl.BlockSpec((1, IN_DIM), lambda i: (0, 0)),
        ],
        out_specs=(
            pl.BlockSpec((_NB, HID), lambda i: (i, 0)),
            pl.BlockSpec((_NB, IN_DIM), lambda i: (i, 0)),
        ),
        out_shape=(
            jax.ShapeDtypeStruct((N, HID), jnp.float32),
            jax.ShapeDtypeStruct((N, IN_DIM), jnp.float32),
        ),
    )(parts, dens, wt, b)


_SCALE_BE = 8192


def _run_scale(gathered, ex_cols):
    grid = EPAD // _SCALE_BE
    return pl.pallas_call(
        _scale_body,
        grid=(grid,),
        in_specs=[
            pl.BlockSpec((_SCALE_BE, R * HID), lambda i: (i, 0)),
            pl.BlockSpec((_SCALE_BE, 2), lambda i: (i, 0)),
        ],
        out_specs=pl.BlockSpec((_SCALE_BE, HID), lambda i: (i, 0)),
        out_shape=jax.ShapeDtypeStruct((EPAD, HID), jnp.float32),
    )(gathered, ex_cols)


@functools.partial(
    pl.kernel,
    mesh=_mesh,
    compiler_params=_sc_params,
    out_type=(
        jax.ShapeDtypeStruct((ROWS, 128), jnp.float32),
        jax.ShapeDtypeStruct((2, NPAD), jnp.float32),
    ),
    scratch_types=[
        pltpu.VMEM((SROWS, 128), jnp.float32),
        pltpu.VMEM((SROWS, 128), jnp.float32),
        pltpu.VMEM((UROWS, 128), jnp.float32),
        pltpu.VMEM((ROWS_PER_TILE, 128), jnp.int32),
        pltpu.VMEM((ROWS_PER_TILE, 128), jnp.int32),
        pltpu.VMEM((ROWS_PER_TILE, 128), jnp.int32),
        pltpu.VMEM((ROWS_PER_TILE, 128), jnp.float32),
        pltpu.VMEM((NPAD,), jnp.float32),
        pltpu.VMEM((16, NSLICE), jnp.float32),
        pltpu.VMEM_SHARED((16, NPAD), jnp.float32),
        pltpu.VMEM_SHARED((2 * SROWS + UROWS, 128), jnp.float32),
        pltpu.SemaphoreType.DMA,
    ],
)
def _sc_scores(sq_h, sk_h, u_h, iq_h, ik_h, dd_h,
               ex_h, den_h,
               sq_v, sk_v, u_v, iq_v, ik_v, dd_v, ex_v,
               priv, red_v, stage, tstage, ld_sem):
    c = lax.axis_index("c")
    s = lax.axis_index("s")
    wid = s * 2 + c

    base = wid * ROWS_PER_TILE

    @pl.when(s == 0)
    def _():
        pltpu.sync_copy(sq_h, tstage.at[pl.ds(0, SROWS)])
        pltpu.sync_copy(sk_h, tstage.at[pl.ds(SROWS, SROWS)])
        pltpu.sync_copy(u_h, tstage.at[pl.ds(2 * SROWS, UROWS)])

    zv = jnp.zeros((16,), jnp.float32)

    @pl.loop(0, NPAD // 16)
    def _(v):
        priv[pl.ds(v * 16, 16)] = zv

    lds = [
        pltpu.async_copy(iq_h.at[pl.ds(base, ROWS_PER_TILE)], iq_v, ld_sem),
        pltpu.async_copy(ik_h.at[pl.ds(base, ROWS_PER_TILE)], ik_v, ld_sem),
        pltpu.async_copy(dd_h.at[pl.ds(base, ROWS_PER_TILE)], dd_v, ld_sem),
    ]
    plsc.subcore_barrier()
    pltpu.sync_copy(tstage.at[pl.ds(0, SROWS)], sq_v)
    pltpu.sync_copy(tstage.at[pl.ds(SROWS, SROWS)], sk_v)
    pltpu.sync_copy(tstage.at[pl.ds(2 * SROWS, UROWS)], u_v)
    for h in lds:
        h.wait()

    @pl.loop(0, ROWS_PER_TILE)
    def _(j):
        @pl.loop(0, 8)
        def _(v):
            sl = pl.ds(v * 16, 16)
            dd = dd_v[j, sl]
            iq = iq_v[j, sl]
            ik = ik_v[j, sl]
            a = plsc.load_gather(
                sq_v, [lax.shift_right_logical(iq, 7), iq & 127])
            b = plsc.load_gather(
                sk_v, [lax.shift_right_logical(ik, 7), ik & 127])
            u = plsc.load_gather(
                u_v, [lax.shift_right_logical(dd, 7), dd & 127])
            e = jnp.exp(_lrelu(a + b) - u)
            ex_v[j, sl] = e
            plsc.addupdate_scatter(priv, [dd], e)

    ex_out = pltpu.async_copy(
        ex_v, ex_h.at[pl.ds(base, ROWS_PER_TILE)], ld_sem)
    pltpu.sync_copy(priv, stage.at[s])
    plsc.subcore_barrier()

    pltpu.sync_copy(stage.at[:, pl.ds(s * NSLICE, NSLICE)], red_v)

    @pl.loop(0, NSLICE // 16)
    def _(v):
        sl = pl.ds(v * 16, 16)
        acc = red_v[0, sl]
        for j in range(1, 16):
            acc = acc + red_v[j, sl]
        red_v[0, sl] = acc

    pltpu.sync_copy(red_v.at[0], den_h.at[c, pl.ds(s * NSLICE, NSLICE)])
    ex_out.wait()


@functools.partial(
    pl.kernel,
    mesh=_mesh,
    compiler_params=_sc_params,
    out_type=jax.ShapeDtypeStruct((EPAD, R * HID), jnp.float32),
    scratch_types=[
        pltpu.VMEM((ROWS_PER_TILE, 128), jnp.int32),
        pltpu.VMEM((256, R * HID), jnp.float32),
        pltpu.VMEM((256, R * HID), jnp.float32),
        pltpu.SemaphoreType.DMA,
        pltpu.SemaphoreType.DMA,
        pltpu.SemaphoreType.DMA,
        pltpu.SemaphoreType.DMA,
    ],
)
def _sc_gather(xw_h, ik_h, g_h, ik_v, rows_a, rows_b,
               g_sem_a, g_sem_b, w_sem_a, w_sem_b):
    c = lax.axis_index("c")
    s = lax.axis_index("s")
    wid = s * 2 + c
    base = wid * ROWS_PER_TILE
    pltpu.sync_copy(ik_h.at[pl.ds(base, ROWS_PER_TILE)], ik_v)

    bufs = (rows_a, rows_b)
    g_sems = (g_sem_a, g_sem_b)
    w_sems = (w_sem_a, w_sem_b)
    n_chunks = ROWS_PER_TILE // 2
    w_hs = [None, None]
    for t in range(n_chunks):
        b = t % 2
        if w_hs[b] is not None:
            w_hs[b].wait()
        g_hs = [
            pltpu.async_copy(
                xw_h.at[ik_v.at[t * 2 + j]],
                bufs[b].at[pl.ds(j * 128, 128)],
                g_sems[b])
            for j in range(2)
        ]
        for h in g_hs:
            h.wait()
        w_hs[b] = pltpu.async_copy(
            bufs[b], g_h.at[pl.ds((base + t * 2) * 128, 256)], w_sems[b])
    for h in w_hs:
        if h is not None:
            h.wait()


@functools.partial(
    pl.kernel,
    mesh=_mesh,
    compiler_params=_sc_lin_params,
    out_type=jax.ShapeDtypeStruct((2, NPAD, HID), jnp.float32),
    scratch_types=[
        pltpu.VMEM((ROWS_PER_TILE, 128), jnp.int32),
        pltpu.VMEM((512, HID), jnp.float32),
        pltpu.VMEM((512, HID), jnp.float32),
        pltpu.VMEM_SHARED((NPAD, HID), jnp.float32),
        pltpu.SemaphoreType.DMA,
        pltpu.SemaphoreType.DMA,
        pltpu.SemaphoreType.DMA,
        pltpu.SemaphoreType.DMA,
    ],
)
def _sc_scatter(sc_h, dd_h, out_h, dd_v, rows_a, rows_b, racc,
                i_sem_a, i_sem_b, s_sem_a, s_sem_b):
    c = lax.axis_index("c")
    s = lax.axis_index("s")
    wid = s * 2 + c

    zv = jnp.zeros((16,), jnp.float32)

    @pl.loop(0, 512)
    def _(r):
        for cc in range(HID // 16):
            rows_a[r, pl.ds(cc * 16, 16)] = zv

    pltpu.sync_copy(rows_a, racc.at[pl.ds(s * 640, 512)])
    pltpu.sync_copy(rows_a.at[pl.ds(0, 128)],
                    racc.at[pl.ds(s * 640 + 512, 128)])
    plsc.subcore_barrier()

    base = wid * ROWS_PER_TILE
    pltpu.sync_copy(dd_h.at[pl.ds(base, ROWS_PER_TILE)], dd_v)

    bufs = (rows_a, rows_b)
    i_sems = (i_sem_a, i_sem_b)
    s_sems = (s_sem_a, s_sem_b)
    n_chunks = ROWS_PER_TILE // 4

    def start_in(t):
        b = t % 2
        return pltpu.async_copy(
            sc_h.at[pl.ds((base + t * 4) * 128, 512)], bufs[b], i_sems[b])

    in_hs = [start_in(0), None]
    sc_hs = [[], []]
    for t in range(n_chunks):
        b = t % 2
        if t + 1 < n_chunks:
            nb = (t + 1) % 2
            for h in sc_hs[nb]:
                h.wait()
            sc_hs[nb] = []
            in_hs[nb] = start_in(t + 1)
        in_hs[b].wait()
        sc_hs[b] = [
            pltpu.async_copy(
                bufs[b].at[pl.ds(j * 128, 128)],
                racc.at[dd_v.at[t * 4 + j]],
                s_sems[b], add=True)
            for j in range(4)
        ]
    for hs in sc_hs:
        for h in hs:
            h.wait()

    plsc.subcore_barrier()

    @pl.when(s == 0)
    def _():
        pltpu.sync_copy(racc, out_h.at[c])


def _layer(xw, s, iqp, ikp, srcp, ddp, tp):
    u = _run_ubound(s).reshape(N)
    u_pad = jnp.concatenate(
        [u, jnp.full((NPAD - N,), 1000.0, jnp.float32)]).reshape(UROWS, 128)

    def to_2d(col):
        flat = s[:, col:col + R].T.reshape(R * N)
        return jnp.pad(flat, (0, SROWS * 128 - R * N)).reshape(SROWS, 128)

    ex, den = _sc_scores(to_2d(0), to_2d(R), u_pad, iqp, ikp, ddp)
    gathered = _sc_gather(xw, srcp)
    exc = ex.reshape(EPAD, 1)
    exc = jnp.concatenate([exc * (1.0 - tp), exc * tp], axis=1)
    scaled = _run_scale(gathered, exc)
    parts = _sc_scatter(scaled, ddp)
    return parts, den.reshape(R, NPAD, 1)


@jax.jit
def kernel(features, edge_index, edge_type, w1, q1, k1, w2, q2, k2,
           dec_w, dec_b):
    src = edge_index[0]
    dst = edge_index[1]
    iq = edge_type * N + dst
    ik = edge_type * N + src

    pad_iota = jax.lax.iota(jnp.int32, EPAD - E)

    def pad_to_rows(a, pad_vals):
        return jnp.concatenate([a, pad_vals]).reshape(ROWS, 128)

    iqp = pad_to_rows(iq, pad_iota % 128)
    ikp = pad_to_rows(ik, pad_iota % 128)
    srcp = pad_to_rows(src, pad_iota % N)
    ddp = pad_to_rows(dst, N + pad_iota % (NPAD - N))
    tp = jnp.pad(edge_type.astype(jnp.float32),
                 (0, EPAD - E)).reshape(EPAD, 1)

    xw1, s1 = _run_head1(features, w1, q1.T, k1.T)
    parts1, den1 = _layer(xw1, s1, iqp, ikp, srcp, ddp, tp)

    xw2, s2 = _run_head2(parts1, den1, w2, q2.T, k2.T)
    parts2, den2 = _layer(xw2, s2, iqp, ikp, srcp, ddp, tp)

    h2, h3 = _run_decoder(parts2, den2, dec_w.T, dec_b.reshape(1, IN_DIM))
    return (h2, h3)
